# Initial kernel scaffold; baseline (speedup 1.0000x reference)
import jax
import jax.numpy as jnp
from jax.experimental import pallas as pl


def kernel(x, edge_index, W_fc, b_fc, W1, b1, W_mu, b_mu, W_ls, b_ls):
    raise NotImplementedError("write your pallas kernel here")


# speedup vs baseline: 9.7508x; 9.7508x over previous
import functools

import jax
import jax.numpy as jnp
from jax import lax
from jax.experimental import pallas as pl
from jax.experimental.pallas import tpu as pltpu
from jax.experimental.pallas import tpu_sc as plsc

N_NODES = 10000
N_EDGES = 320000
N_IN = 128
N_HID = 256
N_LAT = 128

NC = 2
NS = 16
HALF = N_HID // 2
NPAD = 10240
EPT = N_EDGES // NS
CH = 80
ROWS_PT = NPAD // NS
ZR = 128

def _zero_vmem_2d(ref, nrows, ncols):
    def row(r, _):
        def col(j, _):
            ref[r, pl.ds(j * 16, 16)] = jnp.zeros((16,), jnp.float32)
            return 0
        return lax.fori_loop(0, ncols // 16, col, 0)
    lax.fori_loop(0, nrows, row, 0)


def _deg_body(dst_hbm, out_hbm, idx_v, ones_v, zb_v, dacc):
    c = lax.axis_index("c")
    s = lax.axis_index("s")
    seg = NPAD // NS

    def zchunk(i, _):
        zb_v[pl.ds(i * 16, 16)] = jnp.zeros((16,), jnp.float32)
        return 0
    lax.fori_loop(0, seg // 16, zchunk, 0)

    def ochunk(i, _):
        ones_v[pl.ds(i * 16, 16)] = jnp.ones((16,), jnp.float32)
        return 0
    lax.fori_loop(0, CH // 16, ochunk, 0)

    pltpu.sync_copy(zb_v, dacc.at[pl.ds(s * seg, seg)])
    plsc.subcore_barrier()

    epc = N_EDGES // NC
    ept = epc // NS
    base = c * epc + s * ept

    def chunk(j, _):
        off = base + j * CH
        pltpu.sync_copy(dst_hbm.at[pl.ds(off, CH)], idx_v)
        pltpu.sync_copy(ones_v, dacc.at[idx_v], add=True)
        return 0
    lax.fori_loop(0, ept // CH, chunk, 0)

    plsc.subcore_barrier()
    pltpu.sync_copy(dacc.at[pl.ds(s * seg, seg)], out_hbm.at[c, pl.ds(s * seg, seg)])


def _spmm_body(src_hbm, dst_hbm, ta_hbm, tb_hbm, out_hbm, sidx, didx, rows, zbuf, acc, sem):
    c = lax.axis_index("c")
    s = lax.axis_index("s")

    _zero_vmem_2d(zbuf, ZR, HALF)
    for k in range(ROWS_PT // ZR):
        pltpu.sync_copy(zbuf, acc.at[pl.ds(s * ROWS_PT + k * ZR, ZR)])
    plsc.subcore_barrier()

    base = s * EPT

    def chunk(j, _):
        off = base + j * CH
        pltpu.sync_copy(src_hbm.at[pl.ds(off, CH)], sidx)
        pltpu.sync_copy(dst_hbm.at[pl.ds(off, CH)], didx)

        @pl.when(c == 0)
        def _():
            pltpu.async_copy(ta_hbm.at[sidx], rows, sem).wait()

        @pl.when(c == 1)
        def _():
            pltpu.async_copy(tb_hbm.at[sidx], rows, sem).wait()

        pltpu.sync_copy(rows, acc.at[didx], add=True)
        return 0
    lax.fori_loop(0, EPT // CH, chunk, 0)

    plsc.subcore_barrier()
    for k in range(ROWS_PT // ZR):
        r0 = s * ROWS_PT + k * ZR
        pltpu.sync_copy(acc.at[pl.ds(r0, ZR)], out_hbm.at[c, pl.ds(r0, ZR)])


@functools.lru_cache(maxsize=None)
def _sc_kernels():
    mesh = plsc.VectorSubcoreMesh(
        core_axis_name="c", subcore_axis_name="s", num_cores=NC, num_subcores=NS
    )
    deg = pl.kernel(
        _deg_body,
        out_type=jax.ShapeDtypeStruct((NC, NPAD), jnp.float32),
        mesh=mesh,
        scratch_types=[
            pltpu.VMEM((CH,), jnp.int32),
            pltpu.VMEM((CH,), jnp.float32),
            pltpu.VMEM((NPAD // NS,), jnp.float32),
            pltpu.VMEM_SHARED((NPAD,), jnp.float32),
        ],
    )
    spmm = pl.kernel(
        _spmm_body,
        out_type=jax.ShapeDtypeStruct((NC, NPAD, HALF), jnp.float32),
        mesh=mesh,
        scratch_types=[
            pltpu.VMEM((CH,), jnp.int32),
            pltpu.VMEM((CH,), jnp.int32),
            pltpu.VMEM((CH, HALF), jnp.float32),
            pltpu.VMEM((ZR, HALF), jnp.float32),
            pltpu.VMEM_SHARED((NPAD, HALF), jnp.float32),
            pltpu.SemaphoreType.DMA,
        ],
    )
    return deg, spmm


_RB = 1000


def _k1_body(x_ref, w_ref, b_ref, d0_ref, d1_ref, dinv_ref, ha_ref, hb_ref):
    deg = d0_ref[...] + d1_ref[...] + 1.0
    dinv = lax.rsqrt(deg)
    h = jnp.dot(x_ref[...], w_ref[...], preferred_element_type=jnp.float32)
    h = jnp.maximum(h + b_ref[...], 0.0)
    hp = h * dinv
    dinv_ref[...] = dinv
    ha_ref[...] = hp[:, :HALF]
    hb_ref[...] = hp[:, HALF:]


def _k1(x, w, b, d0, d1):
    g = N_NODES // _RB
    return pl.pallas_call(
        _k1_body,
        grid=(g,),
        in_specs=[
            pl.BlockSpec((_RB, N_IN), lambda i: (i, 0)),
            pl.BlockSpec((N_IN, N_HID), lambda i: (0, 0)),
            pl.BlockSpec((1, N_HID), lambda i: (0, 0)),
            pl.BlockSpec((_RB, 1), lambda i: (i, 0)),
            pl.BlockSpec((_RB, 1), lambda i: (i, 0)),
        ],
        out_specs=[
            pl.BlockSpec((_RB, 1), lambda i: (i, 0)),
            pl.BlockSpec((_RB, HALF), lambda i: (i, 0)),
            pl.BlockSpec((_RB, HALF), lambda i: (i, 0)),
        ],
        out_shape=[
            jax.ShapeDtypeStruct((N_NODES, 1), jnp.float32),
            jax.ShapeDtypeStruct((N_NODES, HALF), jnp.float32),
            jax.ShapeDtypeStruct((N_NODES, HALF), jnp.float32),
        ],
    )(x, w, b, d0, d1)


def _k3_body(sa_ref, sb_ref, ha_ref, hb_ref, dinv_ref, w_ref, b_ref, oa_ref, ob_ref):
    dinv = dinv_ref[...]
    h1 = jnp.concatenate(
        [(sa_ref[...] + ha_ref[...]) * dinv, (sb_ref[...] + hb_ref[...]) * dinv],
        axis=1,
    )
    h2 = jnp.dot(h1, w_ref[...], preferred_element_type=jnp.float32)
    h2 = jnp.maximum(h2 + b_ref[...], 0.0) * dinv
    oa_ref[...] = h2[:, :HALF]
    ob_ref[...] = h2[:, HALF:]


def _k3(sa, sb, ha, hb, dinv, w, b):
    g = N_NODES // _RB
    half_spec = pl.BlockSpec((_RB, HALF), lambda i: (i, 0))
    return pl.pallas_call(
        _k3_body,
        grid=(g,),
        in_specs=[
            half_spec, half_spec, half_spec, half_spec,
            pl.BlockSpec((_RB, 1), lambda i: (i, 0)),
            pl.BlockSpec((N_HID, N_HID), lambda i: (0, 0)),
            pl.BlockSpec((1, N_HID), lambda i: (0, 0)),
        ],
        out_specs=[half_spec, half_spec],
        out_shape=[
            jax.ShapeDtypeStruct((N_NODES, HALF), jnp.float32),
            jax.ShapeDtypeStruct((N_NODES, HALF), jnp.float32),
        ],
    )(sa, sb, ha, hb, dinv, w, b)


def _k4_body(sa_ref, sb_ref, ha_ref, hb_ref, dinv_ref, wm_ref, bm_ref,
             wl_ref, bl_ref, mu_ref, ls_ref):
    dinv = dinv_ref[...]
    h2 = jnp.concatenate(
        [(sa_ref[...] + ha_ref[...]) * dinv, (sb_ref[...] + hb_ref[...]) * dinv],
        axis=1,
    )
    mu_ref[...] = jnp.dot(h2, wm_ref[...], preferred_element_type=jnp.float32) + bm_ref[...]
    ls_ref[...] = jnp.dot(h2, wl_ref[...], preferred_element_type=jnp.float32) + bl_ref[...]


def _k4(sa, sb, ha, hb, dinv, wm, bm, wl, bl):
    g = N_NODES // _RB
    half_spec = pl.BlockSpec((_RB, HALF), lambda i: (i, 0))
    lat_spec = pl.BlockSpec((_RB, N_LAT), lambda i: (i, 0))
    return pl.pallas_call(
        _k4_body,
        grid=(g,),
        in_specs=[
            half_spec, half_spec, half_spec, half_spec,
            pl.BlockSpec((_RB, 1), lambda i: (i, 0)),
            pl.BlockSpec((N_HID, N_LAT), lambda i: (0, 0)),
            pl.BlockSpec((1, N_LAT), lambda i: (0, 0)),
            pl.BlockSpec((N_HID, N_LAT), lambda i: (0, 0)),
            pl.BlockSpec((1, N_LAT), lambda i: (0, 0)),
        ],
        out_specs=[lat_spec, lat_spec],
        out_shape=[
            jax.ShapeDtypeStruct((N_NODES, N_LAT), jnp.float32),
            jax.ShapeDtypeStruct((N_NODES, N_LAT), jnp.float32),
        ],
    )(sa, sb, ha, hb, dinv, wm, bm, wl, bl)


def kernel(x, edge_index, W_fc, b_fc, W1, b1, W_mu, b_mu, W_ls, b_ls):
    src = edge_index[0]
    dst = edge_index[1]
    _deg_sc, _spmm_sc = _sc_kernels()

    d = _deg_sc(dst)
    d0 = d[0, :N_NODES].reshape(N_NODES, 1)
    d1 = d[1, :N_NODES].reshape(N_NODES, 1)

    dinv, hpa, hpb = _k1(x, W_fc, b_fc.reshape(1, -1), d0, d1)
    s1 = _spmm_sc(src, dst, hpa, hpb)
    h2pa, h2pb = _k3(s1[0, :N_NODES], s1[1, :N_NODES], hpa, hpb, dinv,
                     W1, b1.reshape(1, -1))
    s2 = _spmm_sc(src, dst, h2pa, h2pb)
    mu, ls = _k4(s2[0, :N_NODES], s2[1, :N_NODES], h2pa, h2pb, dinv,
                 W_mu, b_mu.reshape(1, -1), W_ls, b_ls.reshape(1, -1))
    return (mu, ls)
